# Initial kernel scaffold; baseline (speedup 1.0000x reference)
import jax
import jax.numpy as jnp
from jax.experimental import pallas as pl


def kernel(inp, gate, weight):
    raise NotImplementedError("write your pallas kernel here")


# speedup vs baseline: 1.1934x; 1.1934x over previous
import functools

import jax
import jax.numpy as jnp
from jax import lax
from jax.experimental import pallas as pl
from jax.experimental.pallas import tpu as pltpu
from jax.experimental.pallas import tpu_sc as plsc

TOKENS = 2048
IN_FEAT = 1024
OUT_FEAT = 1024
NUM_EXPERT = 8

TILE = 128
NT = TOKENS // TILE
NS = NT + NUM_EXPERT - 1

NW = 32
ROWS_PER_W = TOKENS // NW


def _routing_metadata(gate):
    g = gate.astype(jnp.int32)
    onehot = (g[:, None] == jnp.arange(NUM_EXPERT, dtype=jnp.int32)[None, :]).astype(jnp.int32)
    cum = jnp.cumsum(onehot, axis=0)
    counts = cum[-1]
    offsets = jnp.concatenate([jnp.zeros((1,), jnp.int32),
                               jnp.cumsum(counts)[:-1].astype(jnp.int32)])
    rank = jnp.take_along_axis(cum, g[:, None], axis=1)[:, 0] - 1
    pos = offsets[g] + rank
    sort_idx = jnp.zeros((TOKENS,), jnp.int32).at[pos].set(
        jnp.arange(TOKENS, dtype=jnp.int32))

    t_ids = jnp.repeat(jnp.arange(NT, dtype=jnp.int32), NUM_EXPERT)
    e_ids = jnp.tile(jnp.arange(NUM_EXPERT, dtype=jnp.int32), NT)
    seg_lo = offsets[e_ids]
    seg_hi = seg_lo + counts[e_ids]
    lo = jnp.maximum(t_ids * TILE, seg_lo)
    hi = jnp.minimum(t_ids * TILE + TILE, seg_hi)
    valid = hi > lo
    key = jnp.where(valid, t_ids * NUM_EXPERT + e_ids, jnp.int32(NT * NUM_EXPERT))
    order = jnp.argsort(key)
    k_valid = jnp.sum(valid.astype(jnp.int32))
    take = jnp.minimum(jnp.arange(NS, dtype=jnp.int32), k_valid - 1)
    step_tile = t_ids[order][take]
    step_eid = e_ids[order][take]
    in_range = jnp.arange(NS, dtype=jnp.int32) < k_valid
    step_lo = jnp.where(in_range, lo[order][take], 0)
    step_hi = jnp.where(in_range, hi[order][take], 0)
    step_first = jnp.concatenate([jnp.ones((1,), jnp.int32),
                                  (step_tile[1:] != step_tile[:-1]).astype(jnp.int32)])
    return pos, sort_idx, step_tile, step_eid, step_lo, step_hi, step_first


def _sc_gather(table, idx):
    mesh = plsc.VectorSubcoreMesh(core_axis_name="c", subcore_axis_name="s")

    @functools.partial(
        pl.kernel, mesh=mesh,
        out_type=jax.ShapeDtypeStruct(table.shape, table.dtype),
        scratch_types=[
            pltpu.VMEM((ROWS_PER_W,), jnp.int32),
            pltpu.VMEM((ROWS_PER_W, table.shape[1]), table.dtype),
            pltpu.SemaphoreType.DMA,
        ],
    )
    def k(table_hbm, idx_hbm, out_hbm, idx_v, rows_v, sem):
        wid = lax.axis_index("s") * 2 + lax.axis_index("c")
        base = wid * ROWS_PER_W
        pltpu.sync_copy(idx_hbm.at[pl.ds(base, ROWS_PER_W)], idx_v)
        pltpu.async_copy(table_hbm.at[idx_v], rows_v, sem).wait()
        pltpu.sync_copy(rows_v, out_hbm.at[pl.ds(base, ROWS_PER_W)])

    return k(table, idx)


def _mm_body(tile_ref, eid_ref, lo_ref, hi_ref, first_ref, x_ref, w_ref, o_ref):
    s = pl.program_id(0)
    row = tile_ref[s] * TILE + lax.broadcasted_iota(jnp.int32, (TILE, 1), 0)
    mask = (row >= lo_ref[s]) & (row < hi_ref[s])
    xm = jnp.where(mask, x_ref[...], 0.0)
    contrib = lax.dot_general(xm, w_ref[0], (((1,), (1,)), ((), ())),
                              preferred_element_type=jnp.float32)

    @pl.when(first_ref[s] == 1)
    def _():
        o_ref[...] = contrib

    @pl.when(first_ref[s] == 0)
    def _():
        o_ref[...] += contrib


def _grouped_matmul(xs, weight, step_tile, step_eid, step_lo, step_hi, step_first):
    grid_spec = pltpu.PrefetchScalarGridSpec(
        num_scalar_prefetch=5,
        grid=(NS,),
        in_specs=[
            pl.BlockSpec((TILE, IN_FEAT), lambda s, t, e, lo, hi, f: (t[s], 0)),
            pl.BlockSpec((1, OUT_FEAT, IN_FEAT), lambda s, t, e, lo, hi, f: (e[s], 0, 0)),
        ],
        out_specs=pl.BlockSpec((TILE, OUT_FEAT), lambda s, t, e, lo, hi, f: (t[s], 0)),
    )
    return pl.pallas_call(
        _mm_body,
        grid_spec=grid_spec,
        out_shape=jax.ShapeDtypeStruct((TOKENS, OUT_FEAT), jnp.float32),
        compiler_params=pltpu.CompilerParams(dimension_semantics=("arbitrary",)),
    )(step_tile, step_eid, step_lo, step_hi, step_first, xs, weight)


def kernel(inp, gate, weight):
    pos, sort_idx, step_tile, step_eid, step_lo, step_hi, step_first = (
        _routing_metadata(gate))
    xs = _sc_gather(inp, sort_idx)
    ys = _grouped_matmul(xs, weight, step_tile, step_eid, step_lo, step_hi,
                         step_first)
    return _sc_gather(ys, pos)
